# baseline (speedup 1.0000x reference)
import functools

import jax
import jax.numpy as jnp
from jax import lax
from jax.experimental import pallas as pl
from jax.experimental.pallas import tpu as pltpu
from jax.experimental.pallas import tpu_sc as plsc

C = 6
NUM_USERS = 100000
NUM_ITEMS = 100000
HIDDEN = 64
BATCH = 16384

NC, NS, L = 2, 16, 16
NW = NC * NS
BPW = BATCH // NW
NCHUNK = 4
CHUNK = BPW // NCHUNK
NBLK = BPW // L


def _body(u_hbm, i_hbm, p_hbm, q_hbm, c_hbm, d_hbm, r_out, p_out,
          uidx, iidx, uadj, iadj, p_rows, q_rows, r_acc, p_acc,
          c_vmem, d_vmem, sem):
  wid = lax.axis_index("s") * NC + lax.axis_index("c")
  base = wid * BPW

  for j in range(NCHUNK):
    pltpu.sync_copy(u_hbm.at[pl.ds(base + j * CHUNK, CHUNK)], uidx.at[j])
    pltpu.sync_copy(i_hbm.at[pl.ds(base + j * CHUNK, CHUNK)], iidx.at[j])
  pltpu.sync_copy(c_hbm, c_vmem)
  pltpu.sync_copy(d_hbm, d_vmem)

  for comp in range(C):
    for j in range(NCHUNK):
      for k in range(CHUNK // L):
        sl = pl.ds(k * L, L)
        uadj[comp, j, sl] = uidx[j, sl] + comp * NUM_USERS
        iadj[comp, j, sl] = iidx[j, sl] + comp * NUM_ITEMS

  iota = lax.iota(jnp.int32, L)

  for comp in range(C):
    copies = []
    for j in range(NCHUNK):
      dst = pl.ds(j * CHUNK, CHUNK)
      copies.append(pltpu.async_copy(p_hbm.at[uadj.at[comp, j]],
                                     p_rows.at[dst], sem))
      copies.append(pltpu.async_copy(q_hbm.at[iadj.at[comp, j]],
                                     q_rows.at[dst], sem))
    for cp in copies:
      cp.wait()

    def blk_body(blk, carry, comp=comp):
      row = blk * L + iota
      racc = jnp.zeros((L,), jnp.float32)
      pacc = jnp.zeros((L,), jnp.float32)
      for hc in range(HIDDEN // L):
        c_chunk = c_vmem[pl.ds(comp * HIDDEN + hc * L, L)]
        for hl in range(L):
          h = hc * L + hl
          col = jnp.full((L,), h, jnp.int32)
          pv = plsc.load_gather(p_rows, [row, col])
          qv = plsc.load_gather(q_rows, [row, col])
          racc = racc + pv * qv
          pacc = pacc + qv * c_chunk[hl]
      d_chunk = d_vmem[pl.ds(0, L)]
      z = pacc + d_chunk[comp]
      pop = 1.0 / (1.0 + jnp.exp(-z))
      sl = pl.ds(blk * L, L)
      if comp == 0:
        r_acc[sl] = racc
        p_acc[sl] = pop
      else:
        r_acc[sl] = r_acc[sl] + racc
        p_acc[sl] = p_acc[sl] + pop
      return carry

    lax.fori_loop(0, NBLK, blk_body, None)

  inv = jnp.float32(1.0 / C)
  for k in range(NBLK):
    sl = pl.ds(k * L, L)
    r_acc[sl] = r_acc[sl] * inv
    p_acc[sl] = p_acc[sl] * inv

  pltpu.sync_copy(r_acc, r_out.at[pl.ds(base, BPW)])
  pltpu.sync_copy(p_acc, p_out.at[pl.ds(base, BPW)])


@jax.jit
def _run(u_batch, i_batch, p_flat, q_flat, c_flat, d_flat):
  mesh = plsc.VectorSubcoreMesh(core_axis_name="c", subcore_axis_name="s",
                                num_cores=NC, num_subcores=NS)
  f = pl.kernel(
      _body,
      out_type=[jax.ShapeDtypeStruct((BATCH,), jnp.float32),
                jax.ShapeDtypeStruct((BATCH,), jnp.float32)],
      mesh=mesh,
      compiler_params=pltpu.CompilerParams(needs_layout_passes=False,
                                           use_tc_tiling_on_sc=False),
      scratch_types=[
          pltpu.VMEM((NCHUNK, CHUNK), jnp.int32),
          pltpu.VMEM((NCHUNK, CHUNK), jnp.int32),
          pltpu.VMEM((C, NCHUNK, CHUNK), jnp.int32),
          pltpu.VMEM((C, NCHUNK, CHUNK), jnp.int32),
          pltpu.VMEM((BPW, HIDDEN), jnp.float32),
          pltpu.VMEM((BPW, HIDDEN), jnp.float32),
          pltpu.VMEM((BPW,), jnp.float32),
          pltpu.VMEM((BPW,), jnp.float32),
          pltpu.VMEM((C * HIDDEN,), jnp.float32),
          pltpu.VMEM((L,), jnp.float32),
          pltpu.SemaphoreType.DMA,
      ],
  )
  return f(u_batch, i_batch, p_flat, q_flat, c_flat, d_flat)


def kernel(u_batch, i_batch, P, Q, c, d):
  p_flat = P.reshape(C * NUM_USERS, HIDDEN)
  q_flat = Q.reshape(C * NUM_ITEMS, HIDDEN)
  c_flat = c.reshape(C * HIDDEN)
  d_flat = jnp.pad(d.reshape(C), (0, L - C))
  r, p = _run(u_batch, i_batch, p_flat, q_flat, c_flat, d_flat)
  return (r.reshape(-1, 1), p.reshape(-1, 1))


# speedup vs baseline: 1.0027x; 1.0027x over previous
import functools

import jax
import jax.numpy as jnp
from jax import lax
from jax.experimental import pallas as pl
from jax.experimental.pallas import tpu as pltpu
from jax.experimental.pallas import tpu_sc as plsc

C = 6
NUM_USERS = 100000
NUM_ITEMS = 100000
HIDDEN = 64
BATCH = 16384

NC, NS, L = 2, 16, 16
NW = NC * NS
BPW = BATCH // NW
NCHUNK = 4
CHUNK = BPW // NCHUNK
NBLK = BPW // L


def _body(u_hbm, i_hbm, p_hbm, q_hbm, c_hbm, d_hbm, r_out, p_out,
          uidx, iidx, p_rows, q_rows, r_acc, p_acc,
          c_vmem, d_vmem, sem):
  wid = lax.axis_index("s") * NC + lax.axis_index("c")
  base = wid * BPW

  for j in range(NCHUNK):
    pltpu.sync_copy(u_hbm.at[pl.ds(base + j * CHUNK, CHUNK)], uidx.at[j])
    pltpu.sync_copy(i_hbm.at[pl.ds(base + j * CHUNK, CHUNK)], iidx.at[j])
  pltpu.sync_copy(c_hbm, c_vmem)
  pltpu.sync_copy(d_hbm, d_vmem)

  iota = lax.iota(jnp.int32, L)

  for comp in range(C):
    copies = []
    for j in range(NCHUNK):
      dst = pl.ds(j * CHUNK, CHUNK)
      copies.append(pltpu.async_copy(p_hbm.at[comp].at[uidx.at[j]],
                                     p_rows.at[dst], sem))
      copies.append(pltpu.async_copy(q_hbm.at[comp].at[iidx.at[j]],
                                     q_rows.at[dst], sem))
    for cp in copies:
      cp.wait()

    def blk_body(blk, carry, comp=comp):
      row = blk * L + iota
      racc = jnp.zeros((L,), jnp.float32)
      pacc = jnp.zeros((L,), jnp.float32)
      for hc in range(HIDDEN // L):
        c_chunk = c_vmem[pl.ds(comp * HIDDEN + hc * L, L)]
        for hl in range(L):
          h = hc * L + hl
          col = jnp.full((L,), h, jnp.int32)
          pv = plsc.load_gather(p_rows, [row, col])
          qv = plsc.load_gather(q_rows, [row, col])
          racc = racc + pv * qv
          pacc = pacc + qv * c_chunk[hl]
      d_chunk = d_vmem[pl.ds(0, L)]
      z = pacc + d_chunk[comp]
      pop = 1.0 / (1.0 + jnp.exp(-z))
      sl = pl.ds(blk * L, L)
      if comp == 0:
        r_acc[sl] = racc
        p_acc[sl] = pop
      else:
        r_acc[sl] = r_acc[sl] + racc
        p_acc[sl] = p_acc[sl] + pop
      return carry

    lax.fori_loop(0, NBLK, blk_body, None)

  inv = jnp.float32(1.0 / C)
  for k in range(NBLK):
    sl = pl.ds(k * L, L)
    r_acc[sl] = r_acc[sl] * inv
    p_acc[sl] = p_acc[sl] * inv

  pltpu.sync_copy(r_acc, r_out.at[pl.ds(base, BPW)])
  pltpu.sync_copy(p_acc, p_out.at[pl.ds(base, BPW)])


@jax.jit
def _run(u_batch, i_batch, p_tab, q_tab, c_flat, d_flat):
  mesh = plsc.VectorSubcoreMesh(core_axis_name="c", subcore_axis_name="s",
                                num_cores=NC, num_subcores=NS)
  f = pl.kernel(
      _body,
      out_type=[jax.ShapeDtypeStruct((BATCH,), jnp.float32),
                jax.ShapeDtypeStruct((BATCH,), jnp.float32)],
      mesh=mesh,
      compiler_params=pltpu.CompilerParams(needs_layout_passes=False,
                                           use_tc_tiling_on_sc=False),
      scratch_types=[
          pltpu.VMEM((NCHUNK, CHUNK), jnp.int32),
          pltpu.VMEM((NCHUNK, CHUNK), jnp.int32),
          pltpu.VMEM((BPW, HIDDEN), jnp.float32),
          pltpu.VMEM((BPW, HIDDEN), jnp.float32),
          pltpu.VMEM((BPW,), jnp.float32),
          pltpu.VMEM((BPW,), jnp.float32),
          pltpu.VMEM((C * HIDDEN,), jnp.float32),
          pltpu.VMEM((L,), jnp.float32),
          pltpu.SemaphoreType.DMA,
      ],
  )
  return f(u_batch, i_batch, p_tab, q_tab, c_flat, d_flat)


def kernel(u_batch, i_batch, P, Q, c, d):
  c_flat = c.reshape(C * HIDDEN)
  d_flat = jnp.pad(d.reshape(C), (0, L - C))
  r, p = _run(u_batch, i_batch, P, Q, c_flat, d_flat)
  return (r.reshape(-1, 1), p.reshape(-1, 1))


# speedup vs baseline: 1.1599x; 1.1568x over previous
import jax
import jax.numpy as jnp
from jax import lax
from jax.experimental import pallas as pl
from jax.experimental.pallas import tpu as pltpu
from jax.experimental.pallas import tpu_sc as plsc

C = 6
NUM_USERS = 100000
NUM_ITEMS = 100000
HIDDEN = 64
BATCH = 16384

NC, NS, L = 2, 16, 16
NW = NC * NS
BPW = BATCH // NW
NCHUNK = 4
CHUNK = BPW // NCHUNK
HALF = BPW // 2
BPH = HALF // L
PAIRS = NUM_USERS // 2
W2 = 2 * HIDDEN


def _gather_idx(v16):
  def g(x):
    return x.at[v16].get(mode="promise_in_bounds")
  return g


def _lane_sum(x, perms):
  for p in perms:
    x = x + x.at[p].get(mode="promise_in_bounds")
  return x


def _body(u_hbm, i_hbm, p_hbm, q_hbm, w_hbm, r_out, p_out,
          uidx, iidx, uoff, ioff, p_rows, q_rows, r_acc, p_acc,
          w_vmem, sem):
  wid = lax.axis_index("s") * NC + lax.axis_index("c")
  base = wid * BPW

  for j in range(NCHUNK):
    pltpu.sync_copy(u_hbm.at[pl.ds(base + j * CHUNK, CHUNK)], uidx.at[j])
    pltpu.sync_copy(i_hbm.at[pl.ds(base + j * CHUNK, CHUNK)], iidx.at[j])
  pltpu.sync_copy(w_hbm, w_vmem)

  zero = jnp.zeros((L,), jnp.float32)
  for k in range(BPW // L):
    r_acc[pl.ds(k * L, L)] = zero
    p_acc[pl.ds(k * L, L)] = zero

  iota = lax.iota(jnp.int32, L)
  perms = [iota ^ 1, iota ^ 2, iota ^ 4, iota ^ 8]
  lane_eq = [iota == l for l in range(L)]

  def comp_body(comp, carry):
    for j in range(NCHUNK):
      for k in range(CHUNK // L):
        sl = pl.ds(k * L, L)
        uoff[j, sl] = lax.shift_right_logical(uidx[j, sl], 1) + comp * PAIRS
        ioff[j, sl] = lax.shift_right_logical(iidx[j, sl], 1) + comp * PAIRS

    cch = [w_vmem[pl.ds(comp * HIDDEN + m * L, L)] for m in range(HIDDEN // L)]
    dch = w_vmem[pl.ds(C * HIDDEN, L)]
    dsplat = dch.at[jnp.full((L,), comp, jnp.int32)].get(
        mode="promise_in_bounds")

    for half in range(2):
      copies = []
      for jj in range(2):
        j = half * 2 + jj
        dst = pl.ds(jj * CHUNK, CHUNK)
        copies.append(pltpu.async_copy(p_hbm.at[uoff.at[j]],
                                       p_rows.at[dst], sem))
        copies.append(pltpu.async_copy(q_hbm.at[ioff.at[j]],
                                       q_rows.at[dst], sem))
      for cp in copies:
        cp.wait()

      def blk_body(bi, carry2, half=half, cch=cch, dsplat=dsplat):
        j = half * 2 + bi // 8
        col = (bi % 8) * L
        u_chunk = uidx[j, pl.ds(col, L)]
        i_chunk = iidx[j, pl.ds(col, L)]
        racc = zero
        pacc = dsplat
        for l in range(L):
          row = bi * L + l
          uoffs = lax.shift_left(u_chunk[l] & 1, 6)
          ioffs = lax.shift_left(i_chunk[l] & 1, 6)
          t = None
          s = None
          for m in range(HIDDEN // L):
            pv = p_rows[row, pl.ds(uoffs + m * L, L)]
            qv = q_rows[row, pl.ds(ioffs + m * L, L)]
            t = pv * qv if t is None else t + pv * qv
            s = qv * cch[m] if s is None else s + qv * cch[m]
          t = _lane_sum(t, perms)
          s = _lane_sum(s, perms)
          racc = jnp.where(lane_eq[l], t, racc)
          pacc = jnp.where(lane_eq[l], s + pacc, pacc)
        pop = 1.0 / (1.0 + jnp.exp(-pacc))
        sl = pl.ds(half * HALF + bi * L, L)
        r_acc[sl] = r_acc[sl] + racc
        p_acc[sl] = p_acc[sl] + pop
        return carry2

      lax.fori_loop(0, BPH, blk_body, None)
    return carry

  lax.fori_loop(0, C, comp_body, None)

  inv = jnp.float32(1.0 / C)
  for k in range(BPW // L):
    sl = pl.ds(k * L, L)
    r_acc[sl] = r_acc[sl] * inv
    p_acc[sl] = p_acc[sl] * inv

  pltpu.sync_copy(r_acc, r_out.at[pl.ds(base, BPW)])
  pltpu.sync_copy(p_acc, p_out.at[pl.ds(base, BPW)])


@jax.jit
def _run(u_batch, i_batch, p_pair, q_pair, w_flat):
  mesh = plsc.VectorSubcoreMesh(core_axis_name="c", subcore_axis_name="s",
                                num_cores=NC, num_subcores=NS)
  f = pl.kernel(
      _body,
      out_type=[jax.ShapeDtypeStruct((BATCH,), jnp.float32),
                jax.ShapeDtypeStruct((BATCH,), jnp.float32)],
      mesh=mesh,
      compiler_params=pltpu.CompilerParams(needs_layout_passes=False,
                                           use_tc_tiling_on_sc=True),
      scratch_types=[
          pltpu.VMEM((NCHUNK, CHUNK), jnp.int32),
          pltpu.VMEM((NCHUNK, CHUNK), jnp.int32),
          pltpu.VMEM((NCHUNK, CHUNK), jnp.int32),
          pltpu.VMEM((NCHUNK, CHUNK), jnp.int32),
          pltpu.VMEM((HALF, W2), jnp.float32),
          pltpu.VMEM((HALF, W2), jnp.float32),
          pltpu.VMEM((BPW,), jnp.float32),
          pltpu.VMEM((BPW,), jnp.float32),
          pltpu.VMEM((512,), jnp.float32),
          pltpu.SemaphoreType.DMA,
      ],
  )
  return f(u_batch, i_batch, p_pair, q_pair, w_flat)


def kernel(u_batch, i_batch, P, Q, c, d):
  p_pair = P.reshape(C * PAIRS, W2)
  q_pair = Q.reshape(C * PAIRS, W2)
  w_flat = jnp.concatenate(
      [c.reshape(C * HIDDEN), d.reshape(C),
       jnp.zeros((512 - C * HIDDEN - C,), jnp.float32)])
  r, p = _run(u_batch, i_batch, p_pair, q_pair, w_flat)
  return (r.reshape(-1, 1), p.reshape(-1, 1))


# speedup vs baseline: 1.1601x; 1.0002x over previous
import jax
import jax.numpy as jnp
from jax import lax
from jax.experimental import pallas as pl
from jax.experimental.pallas import tpu as pltpu
from jax.experimental.pallas import tpu_sc as plsc

C = 6
NUM_USERS = 100000
NUM_ITEMS = 100000
HIDDEN = 64
BATCH = 16384

NC, NS, L = 2, 16, 16
NW = NC * NS
BPW = BATCH // NW
NCHUNK = 4
CHUNK = BPW // NCHUNK
NBLK = BPW // L
MH = HIDDEN // L


def _body(u_hbm, i_hbm, p_hbm, q_hbm, w_hbm, r_out, p_out,
          uidx, iidx, uoff, ioff, p_rows, q_rows, r_acc, p_acc,
          w_vmem, sem):
  wid = lax.axis_index("s") * NC + lax.axis_index("c")
  base = wid * BPW

  for j in range(NCHUNK):
    pltpu.sync_copy(u_hbm.at[pl.ds(base + j * CHUNK, CHUNK)], uidx.at[j])
    pltpu.sync_copy(i_hbm.at[pl.ds(base + j * CHUNK, CHUNK)], iidx.at[j])
  pltpu.sync_copy(w_hbm, w_vmem)

  zero = jnp.zeros((L,), jnp.float32)
  for k in range(NBLK):
    r_acc[pl.ds(k * L, L)] = zero
    p_acc[pl.ds(k * L, L)] = zero

  iota = lax.iota(jnp.int32, L)
  perms = [iota ^ 1, iota ^ 2, iota ^ 4, iota ^ 8]
  lane_eq = [iota == l for l in range(L)]

  def comp_body(comp, carry):
    for j in range(NCHUNK):
      for k in range(CHUNK // L):
        sl = pl.ds(k * L, L)
        uoff[j, sl] = lax.shift_right_logical(uidx[j, sl], 1)
        ioff[j, sl] = lax.shift_right_logical(iidx[j, sl], 1)

    cch = [w_vmem[pl.ds(comp * HIDDEN + m * L, L)] for m in range(MH)]
    dch = w_vmem[pl.ds(C * HIDDEN, L)]
    dsplat = dch.at[jnp.full((L,), comp, jnp.int32)].get(
        mode="promise_in_bounds")

    for half in range(2):
      copies = []
      for jj in range(2):
        j = half * 2 + jj
        dst = pl.ds(jj * CHUNK, CHUNK)
        copies.append(pltpu.async_copy(p_hbm.at[comp].at[uoff.at[j]],
                                       p_rows.at[dst], sem))
        copies.append(pltpu.async_copy(q_hbm.at[comp].at[ioff.at[j]],
                                       q_rows.at[dst], sem))
      for cp in copies:
        cp.wait()

      def blk_body(bi, carry2, half=half, cch=cch, dsplat=dsplat):
        j = half * 2 + bi // 8
        col = (bi % 8) * L
        u_chunk = uidx[j, pl.ds(col, L)]
        i_chunk = iidx[j, pl.ds(col, L)]
        racc = zero
        pacc = dsplat
        for l in range(L):
          row = bi * L + l
          uo = lax.shift_left(u_chunk[l] & 1, 6)
          io = lax.shift_left(i_chunk[l] & 1, 6)
          t = None
          s = None
          for m in range(MH):
            pv = p_rows[row, pl.ds(uo + m * L, L)]
            qv = q_rows[row, pl.ds(io + m * L, L)]
            t = pv * qv if t is None else t + pv * qv
            s = qv * cch[m] if s is None else s + qv * cch[m]
          for p in perms:
            t = t + t.at[p].get(mode="promise_in_bounds")
            s = s + s.at[p].get(mode="promise_in_bounds")
          racc = jnp.where(lane_eq[l], t, racc)
          pacc = jnp.where(lane_eq[l], s + pacc, pacc)
        pop = 1.0 / (1.0 + jnp.exp(-pacc))
        sl = pl.ds(half * (BPW // 2) + bi * L, L)
        r_acc[sl] = r_acc[sl] + racc
        p_acc[sl] = p_acc[sl] + pop
        return carry2

      lax.fori_loop(0, NBLK // 2, blk_body, None)
    return carry

  lax.fori_loop(0, C, comp_body, None)

  inv = jnp.float32(1.0 / C)
  for k in range(NBLK):
    sl = pl.ds(k * L, L)
    r_acc[sl] = r_acc[sl] * inv
    p_acc[sl] = p_acc[sl] * inv

  pltpu.sync_copy(r_acc, r_out.at[pl.ds(base, BPW)])
  pltpu.sync_copy(p_acc, p_out.at[pl.ds(base, BPW)])


def _pair_view(x):
  return x.reshape(C, NUM_USERS // 2, 2 * HIDDEN)


@jax.jit
def _run(u_batch, i_batch, p_tab, q_tab, w_flat):
  mesh = plsc.VectorSubcoreMesh(core_axis_name="c", subcore_axis_name="s",
                                num_cores=NC, num_subcores=NS)
  f = pl.kernel(
      _body,
      out_type=[jax.ShapeDtypeStruct((BATCH,), jnp.float32),
                jax.ShapeDtypeStruct((BATCH,), jnp.float32)],
      mesh=mesh,
      compiler_params=pltpu.CompilerParams(needs_layout_passes=False,
                                           use_tc_tiling_on_sc=True),
      scratch_types=[
          pltpu.VMEM((NCHUNK, CHUNK), jnp.int32),
          pltpu.VMEM((NCHUNK, CHUNK), jnp.int32),
          pltpu.VMEM((NCHUNK, CHUNK), jnp.int32),
          pltpu.VMEM((NCHUNK, CHUNK), jnp.int32),
          pltpu.VMEM((BPW // 2, 2 * HIDDEN), jnp.float32),
          pltpu.VMEM((BPW // 2, 2 * HIDDEN), jnp.float32),
          pltpu.VMEM((BPW,), jnp.float32),
          pltpu.VMEM((BPW,), jnp.float32),
          pltpu.VMEM((512,), jnp.float32),
          pltpu.SemaphoreType.DMA,
      ],
  )
  return f(u_batch, i_batch, p_tab, q_tab, w_flat)


def kernel(u_batch, i_batch, P, Q, c, d):
  w_flat = jnp.concatenate(
      [c.reshape(C * HIDDEN), d.reshape(C),
       jnp.zeros((512 - C * HIDDEN - C,), jnp.float32)])
  r, p = _run(u_batch, i_batch, _pair_view(P), _pair_view(Q), w_flat)
  return (r.reshape(-1, 1), p.reshape(-1, 1))
